# Initial kernel scaffold; baseline (speedup 1.0000x reference)
import jax
import jax.numpy as jnp
from jax.experimental import pallas as pl


def kernel(x, embed):
    raise NotImplementedError("write your pallas kernel here")


# speedup vs baseline: 2.2879x; 2.2879x over previous
import functools

import jax
import jax.numpy as jnp
from jax import lax
from jax.experimental import pallas as pl
from jax.experimental.pallas import tpu as pltpu
from jax.experimental.pallas import tpu_sc as plsc


def _dist_argmax_body(x_ref, et_ref, dist_ref, ind_ref):
    d = jnp.dot(x_ref[...], et_ref[...], preferred_element_type=jnp.float32)
    dist_ref[...] = d
    idx = jnp.argmax(d, axis=1).astype(jnp.int32)
    ind_ref[0, 0, :] = idx


@functools.partial(jax.jit, static_argnames=("row_blk",))
def _dist_argmax(flat_x, embed_t, row_blk=256):
    n, d = flat_x.shape
    c = embed_t.shape[1]
    nblk = n // row_blk
    dist, ind3 = pl.pallas_call(
        _dist_argmax_body,
        grid=(nblk,),
        in_specs=[
            pl.BlockSpec((row_blk, d), lambda i: (i, 0)),
            pl.BlockSpec((d, c), lambda i: (0, 0)),
        ],
        out_specs=[
            pl.BlockSpec((row_blk, c), lambda i: (i, 0)),
            pl.BlockSpec((1, 1, row_blk), lambda i: (i, 0, 0)),
        ],
        out_shape=[
            jax.ShapeDtypeStruct((n, c), jnp.float32),
            jax.ShapeDtypeStruct((nblk, 1, row_blk), jnp.int32),
        ],
    )(flat_x, embed_t)
    return dist, ind3.reshape(n)


def _make_sc_gather(v, d, b):
    nc, ns = 2, 16
    nw = nc * ns
    assert b % (8 * nw) == 0 and d % 16 == 0
    b_per_w = b // nw
    mesh = plsc.VectorSubcoreMesh(core_axis_name="c", subcore_axis_name="s")

    @functools.partial(
        pl.kernel,
        mesh=mesh,
        out_type=jax.ShapeDtypeStruct((b, d), jnp.float32),
        scratch_types=[
            pltpu.VMEM((b_per_w,), jnp.int32),
            pltpu.VMEM((b_per_w, d), jnp.float32),
            pltpu.SemaphoreType.DMA,
        ],
        compiler_params=pltpu.CompilerParams(use_tc_tiling_on_sc=False),
    )
    def gather(table_hbm, idx_hbm, out_hbm, idx_v, rows_v, sem):
        wid = lax.axis_index("s") * nc + lax.axis_index("c")
        base = wid * b_per_w
        pltpu.sync_copy(idx_hbm.at[pl.ds(base, b_per_w)], idx_v)
        pltpu.async_copy(table_hbm.at[idx_v], rows_v, sem).wait()
        pltpu.sync_copy(rows_v, out_hbm.at[pl.ds(base, b_per_w)])

    return gather


def kernel(x, embed):
    b, n, d = x.shape
    c = embed.shape[1]
    flat = x.astype(jnp.float32).reshape(b * n, d)
    table = embed[0].astype(jnp.float32)

    dist, ind = _dist_argmax(flat, table.T)
    quantize = _make_sc_gather(c, d, b * n)(table, ind)

    return (
        quantize.reshape(b, n, d),
        ind.reshape(b, n),
        dist.reshape(1, b, n, c),
    )
